# Initial kernel scaffold; baseline (speedup 1.0000x reference)
import jax
import jax.numpy as jnp
from jax.experimental import pallas as pl


def kernel(x, edge_index, W1, b1, W2, b2):
    raise NotImplementedError("write your pallas kernel here")


# speedup vs baseline: 13.6632x; 13.6632x over previous
import functools

import jax
import jax.numpy as jnp
from jax import lax
from jax.experimental import pallas as pl
from jax.experimental.pallas import tpu as pltpu
from jax.experimental.pallas import tpu_sc as plsc

N = 10000
E = 320000
D = 128
H = 256
NC = 2
NS = 16
NW = NC * NS
SLAB = 640
LAST = N - SLAB * (NS - 1)
EPW = E // NW
CHUNK = 80
NCHUNK = EPW // CHUNK

_MESH = plsc.VectorSubcoreMesh(core_axis_name="c", subcore_axis_name="s")


def _writeback(c, s, acc_sh, out_hbm, buf):
    rbase = s * SLAB

    def wb(j, carry):
        rb = rbase + j * CHUNK
        pltpu.sync_copy(acc_sh.at[pl.ds(rb, CHUNK)], buf)
        pltpu.sync_copy(buf, out_hbm.at[c, pl.ds(rb, CHUNK)])
        return carry

    nsteps = jnp.where(s < NS - 1, SLAB // CHUNK, LAST // CHUNK)
    lax.fori_loop(0, nsteps, wb, 0)


def _deg_body(zeros_hbm, ones_hbm, dst_hbm, out_hbm, dst_v, ones_v, acc_sh,
              sem):
    c = lax.axis_index("c")
    s = lax.axis_index("s")

    @pl.when(s == 0)
    def _():
        pltpu.sync_copy(zeros_hbm, acc_sh)

    pltpu.sync_copy(ones_hbm, ones_v)
    plsc.subcore_barrier()

    ebase = (c * NS + s) * EPW

    def body(i, carry):
        pltpu.sync_copy(dst_hbm.at[pl.ds(ebase + i * CHUNK, CHUNK)], dst_v)
        pltpu.sync_copy(ones_v, acc_sh.at[dst_v], add=True)
        return carry

    lax.fori_loop(0, NCHUNK, body, 0)
    plsc.subcore_barrier()
    _writeback(c, s, acc_sh, out_hbm, ones_v)


_deg_kernel = functools.partial(
    pl.kernel,
    out_type=jax.ShapeDtypeStruct((NC, N, D), jnp.float32),
    mesh=_MESH,
    scratch_types=[
        pltpu.VMEM((CHUNK,), jnp.int32),
        pltpu.VMEM((CHUNK, D), jnp.float32),
        pltpu.VMEM_SHARED((N, D), jnp.float32),
        pltpu.SemaphoreType.DMA,
    ],
)(_deg_body)


def _agg_body(y_hbm, src_hbm, dst_hbm, out_hbm, src_v, dst_v, rows_v, acc_sh,
              sem):
    c = lax.axis_index("c")
    s = lax.axis_index("s")

    @pl.when(s == 0)
    def _():
        pltpu.sync_copy(y_hbm, acc_sh)

    plsc.subcore_barrier()

    ebase = (c * NS + s) * EPW

    def body(i, carry):
        b = ebase + i * CHUNK
        pltpu.sync_copy(src_hbm.at[pl.ds(b, CHUNK)], src_v)
        pltpu.sync_copy(dst_hbm.at[pl.ds(b, CHUNK)], dst_v)
        pltpu.async_copy(y_hbm.at[src_v], rows_v, sem).wait()
        pltpu.sync_copy(rows_v, acc_sh.at[dst_v], add=True)
        return carry

    lax.fori_loop(0, NCHUNK, body, 0)
    plsc.subcore_barrier()
    _writeback(c, s, acc_sh, out_hbm, rows_v)


_agg_kernel = functools.partial(
    pl.kernel,
    out_type=jax.ShapeDtypeStruct((NC, N, D), jnp.float32),
    mesh=_MESH,
    scratch_types=[
        pltpu.VMEM((CHUNK,), jnp.int32),
        pltpu.VMEM((CHUNK,), jnp.int32),
        pltpu.VMEM((CHUNK, D), jnp.float32),
        pltpu.VMEM_SHARED((N, D), jnp.float32),
        pltpu.SemaphoreType.DMA,
    ],
)(_agg_body)


_RB = 1000
_GRID = N // _RB


def _dis_block(degp):
    deg = degp[0][:, 0:1] + degp[1][:, 0:1] + 1.0
    return lax.rsqrt(deg)


def _prescale_body(degp_ref, x_ref, y_ref):
    y_ref[...] = _dis_block(degp_ref) * x_ref[...]


def _mid_body(p_ref, y1_ref, degp_ref, w1_ref, b1_ref, w2_ref, y2_ref):
    dis = _dis_block(degp_ref)
    a1 = dis * (p_ref[0] + p_ref[1] - y1_ref[...])
    h = jnp.maximum(
        jnp.dot(a1, w1_ref[...], preferred_element_type=jnp.float32)
        + b1_ref[...], 0.0)
    g = jnp.dot(h, w2_ref[...], preferred_element_type=jnp.float32)
    y2_ref[...] = dis * g


def _fin_body(q_ref, y2_ref, degp_ref, b2_ref, out_ref):
    dis = _dis_block(degp_ref)
    out_ref[...] = dis * (q_ref[0] + q_ref[1] - y2_ref[...]) + b2_ref[...]


def _rows_spec(width):
    return pl.BlockSpec((_RB, width), lambda i: (i, 0))


def _part_spec(width):
    return pl.BlockSpec((NC, _RB, width), lambda i: (0, i, 0))


def _full_spec(r, ccols):
    return pl.BlockSpec((r, ccols), lambda i: (0, 0))


_prescale = pl.pallas_call(
    _prescale_body,
    grid=(_GRID,),
    in_specs=[_part_spec(D), _rows_spec(D)],
    out_specs=_rows_spec(D),
    out_shape=jax.ShapeDtypeStruct((N, D), jnp.float32),
)

_mid = pl.pallas_call(
    _mid_body,
    grid=(_GRID,),
    in_specs=[_part_spec(D), _rows_spec(D), _part_spec(D),
              _full_spec(D, H), _full_spec(1, H), _full_spec(H, D)],
    out_specs=_rows_spec(D),
    out_shape=jax.ShapeDtypeStruct((N, D), jnp.float32),
)

_fin = pl.pallas_call(
    _fin_body,
    grid=(_GRID,),
    in_specs=[_part_spec(D), _rows_spec(D), _part_spec(D), _full_spec(1, D)],
    out_specs=_rows_spec(D),
    out_shape=jax.ShapeDtypeStruct((N, D), jnp.float32),
)


def kernel(x, edge_index, W1, b1, W2, b2):
    src = edge_index[0].astype(jnp.int32)
    dst = edge_index[1].astype(jnp.int32)
    b1r = b1.reshape(1, H)
    b2r = b2.reshape(1, D)
    zeros = jnp.zeros((N, D), jnp.float32)
    ones = jnp.ones((CHUNK, D), jnp.float32)

    degp = _deg_kernel(zeros, ones, dst)
    y1 = _prescale(degp, x)
    p = _agg_kernel(y1, src, dst)
    y2 = _mid(p, y1, degp, W1, b1r, W2)
    q = _agg_kernel(y2, src, dst)
    return _fin(q, y2, degp, b2r)
